# baseline (speedup 1.0000x reference)
import jax
import jax.numpy as jnp
from jax import lax
from jax.experimental import pallas as pl
from jax.experimental.pallas import tpu as pltpu
from jax.experimental.pallas import tpu_sc as plsc

N_NODES = 100000
TYPES = 100
DIM = 128
NW = 32
CHUNK = 128
SLOTS = 25
PER_W = SLOTS * CHUNK
LAST_BASE = (NW - 1) * PER_W
LAST_N = N_NODES - LAST_BASE
LAST_SLOTS = LAST_N // CHUNK
TAIL = LAST_N - LAST_SLOTS * CHUNK
NBUF = 6


def _body(idx_hbm, table_hbm, out_hbm, table_v, idx_v,
          b0, b1, b2, b3, b4, b5, gsem, s0, s1, s2, s3, s4, s5):
    wid = lax.axis_index("s") * 2 + lax.axis_index("c")
    bufs = (b0, b1, b2, b3, b4, b5)
    sems = (s0, s1, s2, s3, s4, s5)

    base = wid * PER_W

    @pl.when(lax.axis_index("s") == 0)
    def _stage_table():
        pltpu.async_copy(table_hbm, table_v, gsem)

    @pl.when(wid < NW - 1)
    def _stage_idx():
        pltpu.sync_copy(idx_hbm.at[pl.ds(base, PER_W)], idx_v)

    @pl.when(wid == NW - 1)
    def _stage_idx_last():
        pltpu.sync_copy(idx_hbm.at[pl.ds(base, LAST_N)],
                        idx_v.at[pl.ds(0, LAST_N)])

    @pl.when(lax.axis_index("s") == 0)
    def _wait_table():
        pltpu.make_async_copy(table_hbm, table_v, gsem).wait()
    plsc.subcore_barrier()

    def out_base(j):
        return base + j * CHUNK

    def gather_issue(j, b):
        pltpu.async_copy(
            table_v.at[idx_v.at[pl.ds(j * CHUNK, CHUNK)]], bufs[b], gsem)

    def gather_wait(j, b):
        pltpu.make_async_copy(
            table_v.at[idx_v.at[pl.ds(j * CHUNK, CHUNK)]], bufs[b], gsem).wait()

    def scat_issue(j, b):
        pltpu.async_copy(bufs[b], out_hbm.at[pl.ds(out_base(j), CHUNK)], sems[b])

    def scat_wait(j, b):
        pltpu.make_async_copy(
            bufs[b], out_hbm.at[pl.ds(out_base(j), CHUNK)], sems[b]).wait()

    @pl.when(wid < NW - 1)
    def _main():
        gather_issue(0, 0)
        gather_issue(1, 1)
        for j in range(4):
            gather_issue(j + 2, (j + 2) % NBUF)
            gather_wait(j, j)
            scat_issue(j, j)
        for j in (4, 5):
            scat_wait(j - 4, (j + 2) % NBUF)
            gather_issue(j + 2, (j + 2) % NBUF)
            gather_wait(j, j % NBUF)
            scat_issue(j, j % NBUF)

        def steady(i, carry):
            for b in range(NBUF):
                j = i * NBUF + b
                nb = (b + 2) % NBUF
                scat_wait(j - 4, nb)
                gather_issue(j + 2, nb)
                gather_wait(j, b)
                scat_issue(j, b)
            return carry

        lax.fori_loop(1, 3, steady, 0)

        for j in range(18, 23):
            scat_wait(j - 4, (j + 2) % NBUF)
            gather_issue(j + 2, (j + 2) % NBUF)
            gather_wait(j, j % NBUF)
            scat_issue(j, j % NBUF)
        for j in (23, 24):
            gather_wait(j, j % NBUF)
            scat_issue(j, j % NBUF)
        for j in range(19, 25):
            scat_wait(j, j % NBUF)

    @pl.when(wid == NW - 1)
    def _last():
        for j in range(LAST_SLOTS):
            b = j % 2
            gather_issue(j, b)
            gather_wait(j, b)
            pltpu.async_copy(
                bufs[b], out_hbm.at[pl.ds(out_base(j), CHUNK)], sems[b]).wait()
        toff = LAST_SLOTS * CHUNK
        pltpu.async_copy(
            table_v.at[idx_v.at[pl.ds(toff, TAIL)]],
            b2.at[pl.ds(0, TAIL)], gsem).wait()
        pltpu.async_copy(
            b2.at[pl.ds(0, TAIL)],
            out_hbm.at[pl.ds(base + toff, TAIL)], s2).wait()


def kernel(node_type, table):
    mesh = plsc.VectorSubcoreMesh(core_axis_name="c", subcore_axis_name="s")
    f = pl.kernel(
        _body,
        mesh=mesh,
        out_type=jax.ShapeDtypeStruct((N_NODES, DIM), jnp.float32),
        scratch_types=[
            pltpu.VMEM_SHARED((TYPES, DIM), jnp.float32),
            pltpu.VMEM((PER_W,), jnp.int32),
            *[pltpu.VMEM((CHUNK, DIM), jnp.float32) for _ in range(NBUF)],
            pltpu.SemaphoreType.DMA,
            *[pltpu.SemaphoreType.DMA for _ in range(NBUF)],
        ],
    )
    return f(node_type.astype(jnp.int32), table)


# speedup vs baseline: 1.0055x; 1.0055x over previous
import jax
import jax.numpy as jnp
from jax import lax
from jax.experimental import pallas as pl
from jax.experimental.pallas import tpu as pltpu
from jax.experimental.pallas import tpu_sc as plsc

N_NODES = 100000
TYPES = 100
DIM = 128
NW = 32
CHUNK = 128
SLOTS = 25
PER_W = SLOTS * CHUNK
LAST_BASE = (NW - 1) * PER_W
LAST_N = N_NODES - LAST_BASE
LAST_SLOTS = LAST_N // CHUNK
TAIL = LAST_N - LAST_SLOTS * CHUNK
NBUF = 4


def _body(idx_hbm, table_hbm, out_hbm, table_v, idx_v,
          b0, b1, b2, b3, gsem, s0, s1, s2, s3):
    wid = lax.axis_index("s") * 2 + lax.axis_index("c")
    bufs = (b0, b1, b2, b3)
    sems = (s0, s1, s2, s3)

    base = wid * PER_W

    @pl.when(lax.axis_index("s") == 0)
    def _stage_table():
        pltpu.async_copy(table_hbm, table_v, gsem)

    @pl.when(wid < NW - 1)
    def _stage_idx():
        pltpu.sync_copy(idx_hbm.at[pl.ds(base, PER_W)], idx_v)

    @pl.when(wid == NW - 1)
    def _stage_idx_last():
        pltpu.sync_copy(idx_hbm.at[pl.ds(base, LAST_N)],
                        idx_v.at[pl.ds(0, LAST_N)])

    @pl.when(lax.axis_index("s") == 0)
    def _wait_table():
        pltpu.make_async_copy(table_hbm, table_v, gsem).wait()
    plsc.subcore_barrier()

    def out_base(j):
        return base + j * CHUNK

    def gather_issue(j, b):
        pltpu.async_copy(
            table_v.at[idx_v.at[pl.ds(j * CHUNK, CHUNK)]], bufs[b], gsem)

    def gather_wait(j, b):
        pltpu.make_async_copy(
            table_v.at[idx_v.at[pl.ds(j * CHUNK, CHUNK)]], bufs[b], gsem).wait()

    def scat_issue(j, b):
        pltpu.async_copy(bufs[b], out_hbm.at[pl.ds(out_base(j), CHUNK)], sems[b])

    def scat_wait(j, b):
        pltpu.make_async_copy(
            bufs[b], out_hbm.at[pl.ds(out_base(j), CHUNK)], sems[b]).wait()

    @pl.when(wid < NW - 1)
    def _main():
        gather_issue(0, 0)
        for j in range(3):
            gather_issue(j + 1, j + 1)
            gather_wait(j, j)
            scat_issue(j, j)

        def steady(i, carry):
            for b in range(NBUF):
                j = i * NBUF + b
                nb = (b + 1) % NBUF
                scat_wait(j - 3, nb)
                gather_issue(j + 1, nb)
                gather_wait(j, b)
                scat_issue(j, b)
            return carry

        scat_wait(0, 0)
        gather_issue(4, 0)
        gather_wait(3, 3)
        scat_issue(3, 3)

        lax.fori_loop(1, SLOTS // NBUF, steady, 0)

        gather_wait(SLOTS - 1, 0)
        scat_issue(SLOTS - 1, 0)

        scat_wait(21, 1)
        scat_wait(22, 2)
        scat_wait(23, 3)
        scat_wait(SLOTS - 1, 0)

    @pl.when(wid == NW - 1)
    def _last():
        for j in range(LAST_SLOTS):
            b = j % 2
            gather_issue(j, b)
            gather_wait(j, b)
            pltpu.async_copy(
                bufs[b], out_hbm.at[pl.ds(out_base(j), CHUNK)], sems[b]).wait()
        toff = LAST_SLOTS * CHUNK
        pltpu.async_copy(
            table_v.at[idx_v.at[pl.ds(toff, TAIL)]],
            b2.at[pl.ds(0, TAIL)], gsem).wait()
        pltpu.async_copy(
            b2.at[pl.ds(0, TAIL)],
            out_hbm.at[pl.ds(base + toff, TAIL)], s2).wait()


def kernel(node_type, table):
    mesh = plsc.VectorSubcoreMesh(core_axis_name="c", subcore_axis_name="s")
    f = pl.kernel(
        _body,
        mesh=mesh,
        out_type=jax.ShapeDtypeStruct((N_NODES, DIM), jnp.float32),
        scratch_types=[
            pltpu.VMEM_SHARED((TYPES, DIM), jnp.float32),
            pltpu.VMEM((PER_W,), jnp.int32),
            *[pltpu.VMEM((CHUNK, DIM), jnp.float32) for _ in range(NBUF)],
            pltpu.SemaphoreType.DMA,
            *[pltpu.SemaphoreType.DMA for _ in range(NBUF)],
        ],
    )
    return f(node_type.astype(jnp.int32), table)


# speedup vs baseline: 1.0181x; 1.0125x over previous
import jax
import jax.numpy as jnp
from jax import lax
from jax.experimental import pallas as pl
from jax.experimental.pallas import tpu as pltpu
from jax.experimental.pallas import tpu_sc as plsc

N_NODES = 100000
TYPES = 100
DIM = 128
NW = 32
CHUNK = 128
FULL = 24
BIG_W = 20
PER_BIG = FULL * CHUNK + 56
PER_SMALL = FULL * CHUNK + 48
TOFF = FULL * CHUNK
NBUF = 4


def _body(idx_hbm, table_hbm, out_hbm, table_v, idx_v,
          b0, b1, b2, b3, gsem, s0, s1, s2, s3):
    wid = lax.axis_index("s") * 2 + lax.axis_index("c")
    bufs = (b0, b1, b2, b3)
    sems = (s0, s1, s2, s3)

    base = PER_SMALL * wid + 8 * jnp.minimum(wid, BIG_W)
    is_big = wid < BIG_W

    @pl.when(lax.axis_index("s") == 0)
    def _stage_table():
        pltpu.async_copy(table_hbm, table_v, gsem)

    @pl.when(is_big)
    def _stage_idx_big():
        pltpu.sync_copy(idx_hbm.at[pl.ds(base, PER_BIG)],
                        idx_v.at[pl.ds(0, PER_BIG)])

    @pl.when(jnp.logical_not(is_big))
    def _stage_idx_small():
        pltpu.sync_copy(idx_hbm.at[pl.ds(base, PER_SMALL)],
                        idx_v.at[pl.ds(0, PER_SMALL)])

    @pl.when(lax.axis_index("s") == 0)
    def _wait_table():
        pltpu.make_async_copy(table_hbm, table_v, gsem).wait()
    plsc.subcore_barrier()

    def out_base(j):
        return base + j * CHUNK

    def gather_issue(j, b):
        pltpu.async_copy(
            table_v.at[idx_v.at[pl.ds(j * CHUNK, CHUNK)]], bufs[b], gsem)

    def gather_wait(j, b):
        pltpu.make_async_copy(
            table_v.at[idx_v.at[pl.ds(j * CHUNK, CHUNK)]], bufs[b], gsem).wait()

    def scat_issue(j, b):
        pltpu.async_copy(bufs[b], out_hbm.at[pl.ds(out_base(j), CHUNK)], sems[b])

    def scat_wait(j, b):
        pltpu.make_async_copy(
            bufs[b], out_hbm.at[pl.ds(out_base(j), CHUNK)], sems[b]).wait()

    def tail_both(op):
        @pl.when(is_big)
        def _t_big():
            op(56)
        @pl.when(jnp.logical_not(is_big))
        def _t_small():
            op(48)

    def tail_gather_issue():
        tail_both(lambda n: pltpu.async_copy(
            table_v.at[idx_v.at[pl.ds(TOFF, n)]], b0.at[pl.ds(0, n)], gsem))

    def tail_gather_wait():
        tail_both(lambda n: pltpu.make_async_copy(
            table_v.at[idx_v.at[pl.ds(TOFF, n)]], b0.at[pl.ds(0, n)],
            gsem).wait())

    def tail_scat_issue():
        tail_both(lambda n: pltpu.async_copy(
            b0.at[pl.ds(0, n)], out_hbm.at[pl.ds(base + TOFF, n)], s0))

    def tail_scat_wait():
        tail_both(lambda n: pltpu.make_async_copy(
            b0.at[pl.ds(0, n)], out_hbm.at[pl.ds(base + TOFF, n)], s0).wait())

    gather_issue(0, 0)
    for j in range(3):
        gather_issue(j + 1, j + 1)
        gather_wait(j, j)
        scat_issue(j, j)

    scat_wait(0, 0)
    gather_issue(4, 0)
    gather_wait(3, 3)
    scat_issue(3, 3)

    def steady(i, carry):
        for b in range(NBUF):
            j = i * NBUF + b
            nb = (b + 1) % NBUF
            scat_wait(j - 3, nb)
            gather_issue(j + 1, nb)
            gather_wait(j, b)
            scat_issue(j, b)
        return carry

    lax.fori_loop(1, FULL // NBUF - 1, steady, 0)

    for j in range(20, 24):
        b = j % NBUF
        nb = (b + 1) % NBUF
        scat_wait(j - 3, nb)
        if j < 23:
            gather_issue(j + 1, nb)
        else:
            tail_gather_issue()
        gather_wait(j, b)
        scat_issue(j, b)

    tail_gather_wait()
    tail_scat_issue()

    scat_wait(21, 1)
    scat_wait(22, 2)
    scat_wait(23, 3)
    tail_scat_wait()


def kernel(node_type, table):
    mesh = plsc.VectorSubcoreMesh(core_axis_name="c", subcore_axis_name="s")
    f = pl.kernel(
        _body,
        mesh=mesh,
        out_type=jax.ShapeDtypeStruct((N_NODES, DIM), jnp.float32),
        scratch_types=[
            pltpu.VMEM_SHARED((TYPES, DIM), jnp.float32),
            pltpu.VMEM((PER_BIG,), jnp.int32),
            *[pltpu.VMEM((CHUNK, DIM), jnp.float32) for _ in range(NBUF)],
            pltpu.SemaphoreType.DMA,
            *[pltpu.SemaphoreType.DMA for _ in range(NBUF)],
        ],
    )
    return f(node_type.astype(jnp.int32), table)
